# Initial kernel scaffold; baseline (speedup 1.0000x reference)
import jax
import jax.numpy as jnp
from jax.experimental import pallas as pl


def kernel(x, edge_index, W1, b1, W2, b2):
    raise NotImplementedError("write your pallas kernel here")


# speedup vs baseline: 99.6392x; 99.6392x over previous
import functools

import jax
import jax.numpy as jnp
from jax import lax
from jax.experimental import pallas as pl
from jax.experimental.pallas import tpu as pltpu
from jax.experimental.pallas import tpu_sc as plsc

NN = 50000
EE = 800000
BB = 4
HH = 64
NC, NS = 2, 16
L = 16
N_PAD = 50176
SEG = N_PAD // NS
RW = 128
CHUNK = 2048
RCH = CHUNK // RW
E_PAD = 819200
WPT = E_PAD // NS
RPT = WPT // RW
NCH = WPT // CHUNK


def _sc_body(x_hbm, src_hbm, dst2_hbm, w1_hbm, b1_hbm, w2_hbm, b2_hbm,
             out_hbm,
             srcb, dstb, vals, ones, tmpa, tmpb, tmpc, tmpd,
             w1s, b1s, w2s, b2s,
             hist_sh, dinv_sh, u0_sh, u1_sh, a0_sh, a1_sh):
  c = lax.axis_index("c")
  s = lax.axis_index("s")
  seg = s * SEG

  zeros16 = jnp.zeros((L,), jnp.float32)
  ones16 = jnp.ones((L,), jnp.float32)

  pltpu.sync_copy(w1_hbm, w1s)
  pltpu.sync_copy(b1_hbm, b1s)
  pltpu.sync_copy(w2_hbm, w2s)
  pltpu.sync_copy(b2_hbm, b2s)
  w1vs = [w1s[pl.ds(k * L, L)] for k in range(HH // L)]
  b1vs = [b1s[pl.ds(k * L, L)] for k in range(HH // L)]
  w2vs = [w2s[pl.ds(k * L, L)] for k in range(HH // L)]
  pacc = jnp.zeros((L,), jnp.float32)
  qacc = jnp.zeros((L,), jnp.float32)
  babs = jnp.zeros((L,), jnp.float32)
  for k in range(HH // L):
    pw = w1vs[k] * w2vs[k]
    pacc = pacc + jnp.where(w1vs[k] > 0.0, pw, 0.0)
    qacc = qacc + jnp.where(w1vs[k] < 0.0, pw, 0.0)
    babs = jnp.maximum(babs, jnp.abs(b1vs[k]))
  p_sum = jnp.sum(pacc)
  q_sum = jnp.sum(qacc)
  b1_is_zero = jnp.max(babs) == 0.0
  b2v = b2s[pl.ds(0, L)][0]

  @pl.loop(0, CHUNK // L)
  def _(i):
    ones[pl.ds(i * L, L)] = ones16

  @pl.loop(0, SEG // L)
  def _(i):
    tmpa[pl.ds(i * L, L)] = zeros16

  pltpu.sync_copy(tmpa, hist_sh.at[pl.ds(seg, SEG)])
  pltpu.sync_copy(tmpa, a0_sh.at[pl.ds(seg, SEG)])
  pltpu.sync_copy(tmpa, a1_sh.at[pl.ds(seg, SEG)])
  plsc.subcore_barrier()

  @pl.loop(0, NCH)
  def _(ch):
    roff = s * RPT + ch * RCH
    pltpu.sync_copy(dst2_hbm.at[pl.ds(roff, RCH)], dstb)
    for j in range(RCH):
      pltpu.sync_copy(ones.at[pl.ds(j * RW, RW)], hist_sh.at[dstb.at[j]],
                      add=True)

  plsc.subcore_barrier()

  pltpu.sync_copy(hist_sh.at[pl.ds(seg, SEG)], tmpd)

  @pl.loop(0, SEG // L)
  def _(i):
    deg = tmpd[pl.ds(i * L, L)] + 1.0
    ibits = plsc.bitcast(deg, jnp.int32)
    y = plsc.bitcast(jnp.int32(0x5F3759DF) - (ibits >> 1), jnp.float32)
    half = deg * 0.5
    y = y * (1.5 - half * y * y)
    y = y * (1.5 - half * y * y)
    y = y * (1.5 - half * y * y)
    tmpd[pl.ds(i * L, L)] = y

  pltpu.sync_copy(tmpd, dinv_sh.at[pl.ds(seg, SEG)])

  for smp, u_sh in ((0, u0_sh), (1, u1_sh)):
    bs = 2 * c + smp
    pltpu.sync_copy(x_hbm.at[pl.ds(bs * N_PAD + seg, SEG)], tmpa)

    @pl.loop(0, SEG // L)
    def _(i):
      tmpa[pl.ds(i * L, L)] *= tmpd[pl.ds(i * L, L)]

    pltpu.sync_copy(tmpa, u_sh.at[pl.ds(seg, SEG)])

  plsc.subcore_barrier()

  def edge_pass():
    @pl.loop(0, NCH)
    def _(ch):
      woff = s * WPT + ch * CHUNK
      roff = s * RPT + ch * RCH
      pltpu.sync_copy(src_hbm.at[pl.ds(woff, CHUNK)], srcb)
      pltpu.sync_copy(dst2_hbm.at[pl.ds(roff, RCH)], dstb)
      for u_sh, a_sh in ((u0_sh, a0_sh), (u1_sh, a1_sh)):
        pltpu.sync_copy(u_sh.at[srcb], vals)
        for j in range(RCH):
          pltpu.sync_copy(vals.at[pl.ds(j * RW, RW)], a_sh.at[dstb.at[j]],
                          add=True)

  edge_pass()
  plsc.subcore_barrier()

  def dense(u_sh, a_sh):
    pltpu.sync_copy(a_sh.at[pl.ds(seg, SEG)], tmpa)
    pltpu.sync_copy(u_sh.at[pl.ds(seg, SEG)], tmpb)

    def fast(_):
      @pl.loop(0, SEG // L)
      def _(i):
        dv = tmpd[pl.ds(i * L, L)]
        s1 = dv * (tmpa[pl.ds(i * L, L)] + tmpb[pl.ds(i * L, L)])
        t = s1 * jnp.where(s1 > 0.0, p_sum, q_sum)
        tmpa[pl.ds(i * L, L)] = dv * t

    def full(_):
      @pl.loop(0, SEG // L)
      def _(i):
        dv = tmpd[pl.ds(i * L, L)]
        s1 = dv * (tmpa[pl.ds(i * L, L)] + tmpb[pl.ds(i * L, L)])
        t = jnp.zeros((L,), jnp.float32)
        for k in range(HH // L):
          for j in range(L):
            t = t + jnp.maximum(s1 * w1vs[k][j] + b1vs[k][j], 0.0) * w2vs[k][j]
        tmpa[pl.ds(i * L, L)] = dv * t

    lax.cond(b1_is_zero, fast, full, 0)
    pltpu.sync_copy(tmpa, u_sh.at[pl.ds(seg, SEG)])
    @pl.loop(0, SEG // L)
    def _(i):
      tmpb[pl.ds(i * L, L)] = zeros16

    pltpu.sync_copy(tmpb, a_sh.at[pl.ds(seg, SEG)])

  dense(u0_sh, a0_sh)
  dense(u1_sh, a1_sh)
  plsc.subcore_barrier()

  edge_pass()
  plsc.subcore_barrier()

  for smp, (u_sh, a_sh) in ((0, (u0_sh, a0_sh)), (1, (u1_sh, a1_sh))):
    bs = 2 * c + smp
    pltpu.sync_copy(a_sh.at[pl.ds(seg, SEG)], tmpa)
    pltpu.sync_copy(u_sh.at[pl.ds(seg, SEG)], tmpb)
    pltpu.sync_copy(x_hbm.at[pl.ds(bs * N_PAD + seg, SEG)], tmpc)

    @pl.loop(0, SEG // L)
    def _(i):
      dv = tmpd[pl.ds(i * L, L)]
      g2 = tmpa[pl.ds(i * L, L)]
      v = tmpb[pl.ds(i * L, L)]
      xv = tmpc[pl.ds(i * L, L)]
      tmpa[pl.ds(i * L, L)] = xv + 0.5 * (dv * (g2 + v) + b2v)

    pltpu.sync_copy(tmpa, out_hbm.at[pl.ds(bs * N_PAD + seg, SEG)])


@functools.partial(
    pl.kernel,
    out_type=jax.ShapeDtypeStruct((BB * N_PAD,), jnp.float32),
    mesh=plsc.VectorSubcoreMesh(
        core_axis_name="c", subcore_axis_name="s",
        num_cores=NC, num_subcores=NS),
    compiler_params=pltpu.CompilerParams(needs_layout_passes=False),
    scratch_types=[
        pltpu.VMEM((CHUNK,), jnp.int32),
        pltpu.VMEM((RCH, RW), jnp.int32),
        pltpu.VMEM((CHUNK,), jnp.float32),
        pltpu.VMEM((CHUNK,), jnp.float32),
        pltpu.VMEM((SEG,), jnp.float32),
        pltpu.VMEM((SEG,), jnp.float32),
        pltpu.VMEM((SEG,), jnp.float32),
        pltpu.VMEM((SEG,), jnp.float32),
        pltpu.VMEM((HH,), jnp.float32),
        pltpu.VMEM((HH,), jnp.float32),
        pltpu.VMEM((HH,), jnp.float32),
        pltpu.VMEM((L,), jnp.float32),
        pltpu.VMEM_SHARED((N_PAD,), jnp.float32),
        pltpu.VMEM_SHARED((N_PAD,), jnp.float32),
        pltpu.VMEM_SHARED((N_PAD,), jnp.float32),
        pltpu.VMEM_SHARED((N_PAD,), jnp.float32),
        pltpu.VMEM_SHARED((N_PAD,), jnp.float32),
        pltpu.VMEM_SHARED((N_PAD,), jnp.float32),
    ],
)
def _sc_call(*refs):
  _sc_body(*refs)


def kernel(x, edge_index, W1, b1, W2, b2):
  x_pad = jnp.pad(x.astype(jnp.float32), ((0, 0), (0, N_PAD - NN)))
  src = edge_index[0].astype(jnp.int32)
  dst = edge_index[1].astype(jnp.int32)
  src = jnp.pad(src, (0, E_PAD - EE), constant_values=NN)
  dst = jnp.pad(dst, (0, E_PAD - EE), constant_values=NN)
  dst2 = dst.reshape(E_PAD // RW, RW)
  w1 = W1.reshape(-1).astype(jnp.float32)
  b1v = b1.reshape(-1).astype(jnp.float32)
  w2 = W2.reshape(-1).astype(jnp.float32)
  b2v = jnp.pad(b2.reshape(-1).astype(jnp.float32), (0, L - 1))
  out_flat = _sc_call(x_pad.reshape(-1), src, dst2, w1, b1v, w2, b2v)
  return out_flat.reshape(BB, N_PAD)[:, :NN]


# speedup vs baseline: 114.1967x; 1.1461x over previous
import functools

import jax
import jax.numpy as jnp
from jax import lax
from jax.experimental import pallas as pl
from jax.experimental.pallas import tpu as pltpu
from jax.experimental.pallas import tpu_sc as plsc

NN = 50000
EE = 800000
BB = 4
HH = 64
NC, NS = 2, 16
L = 16
N_PAD = 50176
SEG = N_PAD // NS
RW = 128
CHUNK = 2048
RCH = CHUNK // RW
E_PAD = 819200
WPT = E_PAD // NS
RPT = WPT // RW
NCH = WPT // CHUNK


def _sc_body(x_hbm, src_hbm, dst_hbm, w1_hbm, b1_hbm, w2_hbm, b2_hbm,
             out_hbm,
             srcb, dstb, vals, ones, tmpa, tmpb, tmpc, tmpd,
             w1s, b1s, w2s, b2s,
             hist_sh, dinv_sh, u0_sh, u1_sh, a0_sh, a1_sh):
  c = lax.axis_index("c")
  s = lax.axis_index("s")
  seg = s * SEG

  zeros16 = jnp.zeros((L,), jnp.float32)
  ones16 = jnp.ones((L,), jnp.float32)

  pltpu.sync_copy(w1_hbm, w1s)
  pltpu.sync_copy(b1_hbm, b1s)
  pltpu.sync_copy(w2_hbm, w2s)
  pltpu.sync_copy(b2_hbm, b2s)
  w1vs = [w1s[pl.ds(k * L, L)] for k in range(HH // L)]
  b1vs = [b1s[pl.ds(k * L, L)] for k in range(HH // L)]
  w2vs = [w2s[pl.ds(k * L, L)] for k in range(HH // L)]
  pacc = jnp.zeros((L,), jnp.float32)
  qacc = jnp.zeros((L,), jnp.float32)
  babs = jnp.zeros((L,), jnp.float32)
  for k in range(HH // L):
    pw = w1vs[k] * w2vs[k]
    pacc = pacc + jnp.where(w1vs[k] > 0.0, pw, 0.0)
    qacc = qacc + jnp.where(w1vs[k] < 0.0, pw, 0.0)
    babs = jnp.maximum(babs, jnp.abs(b1vs[k]))
  p_sum = jnp.sum(pacc)
  q_sum = jnp.sum(qacc)
  b1_is_zero = jnp.max(babs) == 0.0
  b2v = b2s[pl.ds(0, L)][0]

  @pl.loop(0, CHUNK // L)
  def _(i):
    ones[pl.ds(i * L, L)] = ones16

  @pl.loop(0, SEG // L)
  def _(i):
    tmpa[pl.ds(i * L, L)] = zeros16

  pltpu.sync_copy(tmpa, hist_sh.at[pl.ds(seg, SEG)])
  pltpu.sync_copy(tmpa, a0_sh.at[pl.ds(seg, SEG)])
  pltpu.sync_copy(tmpa, a1_sh.at[pl.ds(seg, SEG)])
  plsc.subcore_barrier()

  @pl.loop(0, NCH)
  def _(ch):
    woff = s * WPT + ch * CHUNK
    pltpu.sync_copy(dst_hbm.at[pl.ds(woff, CHUNK)], dstb)
    pltpu.sync_copy(ones, hist_sh.at[dstb], add=True)

  plsc.subcore_barrier()

  pltpu.sync_copy(hist_sh.at[pl.ds(seg, SEG)], tmpd)

  @pl.loop(0, SEG // L)
  def _(i):
    deg = tmpd[pl.ds(i * L, L)] + 1.0
    ibits = plsc.bitcast(deg, jnp.int32)
    y = plsc.bitcast(jnp.int32(0x5F3759DF) - (ibits >> 1), jnp.float32)
    half = deg * 0.5
    y = y * (1.5 - half * y * y)
    y = y * (1.5 - half * y * y)
    y = y * (1.5 - half * y * y)
    tmpd[pl.ds(i * L, L)] = y

  pltpu.sync_copy(tmpd, dinv_sh.at[pl.ds(seg, SEG)])

  for smp, u_sh in ((0, u0_sh), (1, u1_sh)):
    bs = 2 * c + smp
    pltpu.sync_copy(x_hbm.at[pl.ds(bs * N_PAD + seg, SEG)], tmpa)

    @pl.loop(0, SEG // L)
    def _(i):
      tmpa[pl.ds(i * L, L)] *= tmpd[pl.ds(i * L, L)]

    pltpu.sync_copy(tmpa, u_sh.at[pl.ds(seg, SEG)])

  plsc.subcore_barrier()

  def edge_pass():
    @pl.loop(0, NCH)
    def _(ch):
      woff = s * WPT + ch * CHUNK
      pltpu.sync_copy(src_hbm.at[pl.ds(woff, CHUNK)], srcb)
      pltpu.sync_copy(dst_hbm.at[pl.ds(woff, CHUNK)], dstb)
      for u_sh, a_sh in ((u0_sh, a0_sh), (u1_sh, a1_sh)):
        pltpu.sync_copy(u_sh.at[srcb], vals)
        pltpu.sync_copy(vals, a_sh.at[dstb], add=True)

  edge_pass()
  plsc.subcore_barrier()

  def dense(u_sh, a_sh):
    pltpu.sync_copy(a_sh.at[pl.ds(seg, SEG)], tmpa)
    pltpu.sync_copy(u_sh.at[pl.ds(seg, SEG)], tmpb)

    def fast(_):
      @pl.loop(0, SEG // L)
      def _(i):
        dv = tmpd[pl.ds(i * L, L)]
        s1 = dv * (tmpa[pl.ds(i * L, L)] + tmpb[pl.ds(i * L, L)])
        t = s1 * jnp.where(s1 > 0.0, p_sum, q_sum)
        tmpa[pl.ds(i * L, L)] = dv * t

    def full(_):
      @pl.loop(0, SEG // L)
      def _(i):
        dv = tmpd[pl.ds(i * L, L)]
        s1 = dv * (tmpa[pl.ds(i * L, L)] + tmpb[pl.ds(i * L, L)])
        t = jnp.zeros((L,), jnp.float32)
        for k in range(HH // L):
          for j in range(L):
            t = t + jnp.maximum(s1 * w1vs[k][j] + b1vs[k][j], 0.0) * w2vs[k][j]
        tmpa[pl.ds(i * L, L)] = dv * t

    lax.cond(b1_is_zero, fast, full, 0)
    pltpu.sync_copy(tmpa, u_sh.at[pl.ds(seg, SEG)])
    @pl.loop(0, SEG // L)
    def _(i):
      tmpb[pl.ds(i * L, L)] = zeros16

    pltpu.sync_copy(tmpb, a_sh.at[pl.ds(seg, SEG)])

  dense(u0_sh, a0_sh)
  dense(u1_sh, a1_sh)
  plsc.subcore_barrier()

  edge_pass()
  plsc.subcore_barrier()

  for smp, (u_sh, a_sh) in ((0, (u0_sh, a0_sh)), (1, (u1_sh, a1_sh))):
    bs = 2 * c + smp
    pltpu.sync_copy(a_sh.at[pl.ds(seg, SEG)], tmpa)
    pltpu.sync_copy(u_sh.at[pl.ds(seg, SEG)], tmpb)
    pltpu.sync_copy(x_hbm.at[pl.ds(bs * N_PAD + seg, SEG)], tmpc)

    @pl.loop(0, SEG // L)
    def _(i):
      dv = tmpd[pl.ds(i * L, L)]
      g2 = tmpa[pl.ds(i * L, L)]
      v = tmpb[pl.ds(i * L, L)]
      xv = tmpc[pl.ds(i * L, L)]
      tmpa[pl.ds(i * L, L)] = xv + 0.5 * (dv * (g2 + v) + b2v)

    pltpu.sync_copy(tmpa, out_hbm.at[pl.ds(bs * N_PAD + seg, SEG)])


@functools.partial(
    pl.kernel,
    out_type=jax.ShapeDtypeStruct((BB * N_PAD,), jnp.float32),
    mesh=plsc.VectorSubcoreMesh(
        core_axis_name="c", subcore_axis_name="s",
        num_cores=NC, num_subcores=NS),
    compiler_params=pltpu.CompilerParams(needs_layout_passes=False),
    scratch_types=[
        pltpu.VMEM((CHUNK,), jnp.int32),
        pltpu.VMEM((CHUNK,), jnp.int32),
        pltpu.VMEM((CHUNK,), jnp.float32),
        pltpu.VMEM((CHUNK,), jnp.float32),
        pltpu.VMEM((SEG,), jnp.float32),
        pltpu.VMEM((SEG,), jnp.float32),
        pltpu.VMEM((SEG,), jnp.float32),
        pltpu.VMEM((SEG,), jnp.float32),
        pltpu.VMEM((HH,), jnp.float32),
        pltpu.VMEM((HH,), jnp.float32),
        pltpu.VMEM((HH,), jnp.float32),
        pltpu.VMEM((L,), jnp.float32),
        pltpu.VMEM_SHARED((N_PAD,), jnp.float32),
        pltpu.VMEM_SHARED((N_PAD,), jnp.float32),
        pltpu.VMEM_SHARED((N_PAD,), jnp.float32),
        pltpu.VMEM_SHARED((N_PAD,), jnp.float32),
        pltpu.VMEM_SHARED((N_PAD,), jnp.float32),
        pltpu.VMEM_SHARED((N_PAD,), jnp.float32),
    ],
)
def _sc_call(*refs):
  _sc_body(*refs)


def kernel(x, edge_index, W1, b1, W2, b2):
  x_pad = jnp.pad(x.astype(jnp.float32), ((0, 0), (0, N_PAD - NN)))
  src = edge_index[0].astype(jnp.int32)
  dst = edge_index[1].astype(jnp.int32)
  src = jnp.pad(src, (0, E_PAD - EE), constant_values=NN)
  dst = jnp.pad(dst, (0, E_PAD - EE), constant_values=NN)
  w1 = W1.reshape(-1).astype(jnp.float32)
  b1v = b1.reshape(-1).astype(jnp.float32)
  w2 = W2.reshape(-1).astype(jnp.float32)
  b2v = jnp.pad(b2.reshape(-1).astype(jnp.float32), (0, L - 1))
  out_flat = _sc_call(x_pad.reshape(-1), src, dst, w1, b1v, w2, b2v)
  return out_flat.reshape(BB, N_PAD)[:, :NN]


# speedup vs baseline: 151.8631x; 1.3298x over previous
import functools

import jax
import jax.numpy as jnp
from jax import lax
from jax.experimental import pallas as pl
from jax.experimental.pallas import tpu as pltpu
from jax.experimental.pallas import tpu_sc as plsc

NN = 50000
EE = 800000
BB = 4
HH = 64
NC, NS = 2, 16
L = 16
N_PAD = 50176
SEG = N_PAD // NS
CHUNK = 6400
E_PAD = 819200
WPT = E_PAD // NS
NCH = WPT // CHUNK


def _sc_body(x_hbm, src_hbm, dst_hbm, w1_hbm, b1_hbm, w2_hbm, b2_hbm,
             out_hbm,
             srcb0, srcb1, srcb2, dstb0, dstb1, dstb2,
             valsa0, valsa1, valsb0, valsb1, ones,
             tmpa, tmpb, tmpc, tmpd,
             w1s, b1s, w2s, b2s,
             sem_pre, sem_g0, sem_g1, sem_s0, sem_s1,
             hist_sh, dinv_sh, u0_sh, u1_sh, a0_sh, a1_sh):
  srcs = [srcb0, srcb1, srcb2]
  dsts = [dstb0, dstb1, dstb2]
  valsa = [valsa0, valsa1]
  valsb = [valsb0, valsb1]
  c = lax.axis_index("c")
  s = lax.axis_index("s")
  seg = s * SEG

  zeros16 = jnp.zeros((L,), jnp.float32)
  ones16 = jnp.ones((L,), jnp.float32)

  pltpu.sync_copy(w1_hbm, w1s)
  pltpu.sync_copy(b1_hbm, b1s)
  pltpu.sync_copy(w2_hbm, w2s)
  pltpu.sync_copy(b2_hbm, b2s)
  w1vs = [w1s[pl.ds(k * L, L)] for k in range(HH // L)]
  b1vs = [b1s[pl.ds(k * L, L)] for k in range(HH // L)]
  w2vs = [w2s[pl.ds(k * L, L)] for k in range(HH // L)]
  pacc = jnp.zeros((L,), jnp.float32)
  qacc = jnp.zeros((L,), jnp.float32)
  babs = jnp.zeros((L,), jnp.float32)
  for k in range(HH // L):
    pw = w1vs[k] * w2vs[k]
    pacc = pacc + jnp.where(w1vs[k] > 0.0, pw, 0.0)
    qacc = qacc + jnp.where(w1vs[k] < 0.0, pw, 0.0)
    babs = jnp.maximum(babs, jnp.abs(b1vs[k]))
  p_sum = jnp.sum(pacc)
  q_sum = jnp.sum(qacc)
  b1_is_zero = jnp.max(babs) == 0.0
  b2v = b2s[pl.ds(0, L)][0]

  @pl.loop(0, CHUNK // L)
  def _(i):
    ones[pl.ds(i * L, L)] = ones16

  @pl.loop(0, SEG // L)
  def _(i):
    tmpa[pl.ds(i * L, L)] = zeros16

  pltpu.sync_copy(tmpa, hist_sh.at[pl.ds(seg, SEG)])
  pltpu.sync_copy(tmpa, a0_sh.at[pl.ds(seg, SEG)])
  pltpu.sync_copy(tmpa, a1_sh.at[pl.ds(seg, SEG)])
  plsc.subcore_barrier()

  hpend = []
  pres = [pltpu.async_copy(dst_hbm.at[pl.ds(s * WPT, CHUNK)], dsts[0],
                           sem_pre)]
  for ch in range(NCH):
    if ch >= 2:
      hpend[ch - 2].wait()
    if ch + 1 < NCH:
      pres.append(pltpu.async_copy(
          dst_hbm.at[pl.ds(s * WPT + (ch + 1) * CHUNK, CHUNK)],
          dsts[(ch + 1) % 3], sem_pre))
    pres[ch].wait()
    hpend.append(pltpu.async_copy(ones, hist_sh.at[dsts[ch % 3]], sem_s0,
                                  add=True))
  hpend[NCH - 2].wait()
  hpend[NCH - 1].wait()
  plsc.subcore_barrier()

  pltpu.sync_copy(hist_sh.at[pl.ds(seg, SEG)], tmpd)

  @pl.loop(0, SEG // L)
  def _(i):
    deg = tmpd[pl.ds(i * L, L)] + 1.0
    ibits = plsc.bitcast(deg, jnp.int32)
    y = plsc.bitcast(jnp.int32(0x5F3759DF) - (ibits >> 1), jnp.float32)
    half = deg * 0.5
    y = y * (1.5 - half * y * y)
    y = y * (1.5 - half * y * y)
    y = y * (1.5 - half * y * y)
    tmpd[pl.ds(i * L, L)] = y

  pltpu.sync_copy(tmpd, dinv_sh.at[pl.ds(seg, SEG)])

  for smp, u_sh in ((0, u0_sh), (1, u1_sh)):
    bs = 2 * c + smp
    pltpu.sync_copy(x_hbm.at[pl.ds(bs * N_PAD + seg, SEG)], tmpa)

    @pl.loop(0, SEG // L)
    def _(i):
      tmpa[pl.ds(i * L, L)] *= tmpd[pl.ds(i * L, L)]

    pltpu.sync_copy(tmpa, u_sh.at[pl.ds(seg, SEG)])

  plsc.subcore_barrier()

  def edge_pass():
    sspend = []
    pres = [(pltpu.async_copy(src_hbm.at[pl.ds(s * WPT, CHUNK)], srcs[0],
                              sem_pre),
             pltpu.async_copy(dst_hbm.at[pl.ds(s * WPT, CHUNK)], dsts[0],
                              sem_pre))]
    for ch in range(NCH):
      p3 = ch % 3
      p2 = ch % 2
      if ch >= 2:
        s0d, s1d = sspend[ch - 2]
        s0d.wait()
        s1d.wait()
      if ch + 1 < NCH:
        woff = s * WPT + (ch + 1) * CHUNK
        q3 = (ch + 1) % 3
        pres.append((pltpu.async_copy(src_hbm.at[pl.ds(woff, CHUNK)],
                                      srcs[q3], sem_pre),
                     pltpu.async_copy(dst_hbm.at[pl.ds(woff, CHUNK)],
                                      dsts[q3], sem_pre)))
      pa, pb = pres[ch]
      pa.wait()
      pb.wait()
      g0 = pltpu.async_copy(u0_sh.at[srcs[p3]], valsa[p2], sem_g0)
      g1 = pltpu.async_copy(u1_sh.at[srcs[p3]], valsb[p2], sem_g1)
      g0.wait()
      s0 = pltpu.async_copy(valsa[p2], a0_sh.at[dsts[p3]], sem_s0, add=True)
      g1.wait()
      s1 = pltpu.async_copy(valsb[p2], a1_sh.at[dsts[p3]], sem_s1, add=True)
      sspend.append((s0, s1))
    for ch in (NCH - 2, NCH - 1):
      s0d, s1d = sspend[ch]
      s0d.wait()
      s1d.wait()

  edge_pass()
  plsc.subcore_barrier()

  def dense(u_sh, a_sh):
    pltpu.sync_copy(a_sh.at[pl.ds(seg, SEG)], tmpa)
    pltpu.sync_copy(u_sh.at[pl.ds(seg, SEG)], tmpb)

    def fast(_):
      @pl.loop(0, SEG // L)
      def _(i):
        dv = tmpd[pl.ds(i * L, L)]
        s1 = dv * (tmpa[pl.ds(i * L, L)] + tmpb[pl.ds(i * L, L)])
        t = s1 * jnp.where(s1 > 0.0, p_sum, q_sum)
        tmpa[pl.ds(i * L, L)] = dv * t

    def full(_):
      @pl.loop(0, SEG // L)
      def _(i):
        dv = tmpd[pl.ds(i * L, L)]
        s1 = dv * (tmpa[pl.ds(i * L, L)] + tmpb[pl.ds(i * L, L)])
        t = jnp.zeros((L,), jnp.float32)
        for k in range(HH // L):
          for j in range(L):
            t = t + jnp.maximum(s1 * w1vs[k][j] + b1vs[k][j], 0.0) * w2vs[k][j]
        tmpa[pl.ds(i * L, L)] = dv * t

    lax.cond(b1_is_zero, fast, full, 0)
    pltpu.sync_copy(tmpa, u_sh.at[pl.ds(seg, SEG)])
    @pl.loop(0, SEG // L)
    def _(i):
      tmpb[pl.ds(i * L, L)] = zeros16

    pltpu.sync_copy(tmpb, a_sh.at[pl.ds(seg, SEG)])

  dense(u0_sh, a0_sh)
  dense(u1_sh, a1_sh)
  plsc.subcore_barrier()

  edge_pass()
  plsc.subcore_barrier()

  for smp, (u_sh, a_sh) in ((0, (u0_sh, a0_sh)), (1, (u1_sh, a1_sh))):
    bs = 2 * c + smp
    pltpu.sync_copy(a_sh.at[pl.ds(seg, SEG)], tmpa)
    pltpu.sync_copy(u_sh.at[pl.ds(seg, SEG)], tmpb)
    pltpu.sync_copy(x_hbm.at[pl.ds(bs * N_PAD + seg, SEG)], tmpc)

    @pl.loop(0, SEG // L)
    def _(i):
      dv = tmpd[pl.ds(i * L, L)]
      g2 = tmpa[pl.ds(i * L, L)]
      v = tmpb[pl.ds(i * L, L)]
      xv = tmpc[pl.ds(i * L, L)]
      tmpa[pl.ds(i * L, L)] = xv + 0.5 * (dv * (g2 + v) + b2v)

    pltpu.sync_copy(tmpa, out_hbm.at[pl.ds(bs * N_PAD + seg, SEG)])


@functools.partial(
    pl.kernel,
    out_type=jax.ShapeDtypeStruct((BB * N_PAD,), jnp.float32),
    mesh=plsc.VectorSubcoreMesh(
        core_axis_name="c", subcore_axis_name="s",
        num_cores=NC, num_subcores=NS),
    compiler_params=pltpu.CompilerParams(needs_layout_passes=False),
    scratch_types=[
        pltpu.VMEM((CHUNK,), jnp.int32),
        pltpu.VMEM((CHUNK,), jnp.int32),
        pltpu.VMEM((CHUNK,), jnp.int32),
        pltpu.VMEM((CHUNK,), jnp.int32),
        pltpu.VMEM((CHUNK,), jnp.int32),
        pltpu.VMEM((CHUNK,), jnp.int32),
        pltpu.VMEM((CHUNK,), jnp.float32),
        pltpu.VMEM((CHUNK,), jnp.float32),
        pltpu.VMEM((CHUNK,), jnp.float32),
        pltpu.VMEM((CHUNK,), jnp.float32),
        pltpu.VMEM((CHUNK,), jnp.float32),
        pltpu.VMEM((SEG,), jnp.float32),
        pltpu.VMEM((SEG,), jnp.float32),
        pltpu.VMEM((SEG,), jnp.float32),
        pltpu.VMEM((SEG,), jnp.float32),
        pltpu.VMEM((HH,), jnp.float32),
        pltpu.VMEM((HH,), jnp.float32),
        pltpu.VMEM((HH,), jnp.float32),
        pltpu.VMEM((L,), jnp.float32),
        pltpu.SemaphoreType.DMA,
        pltpu.SemaphoreType.DMA,
        pltpu.SemaphoreType.DMA,
        pltpu.SemaphoreType.DMA,
        pltpu.SemaphoreType.DMA,
        pltpu.VMEM_SHARED((N_PAD,), jnp.float32),
        pltpu.VMEM_SHARED((N_PAD,), jnp.float32),
        pltpu.VMEM_SHARED((N_PAD,), jnp.float32),
        pltpu.VMEM_SHARED((N_PAD,), jnp.float32),
        pltpu.VMEM_SHARED((N_PAD,), jnp.float32),
        pltpu.VMEM_SHARED((N_PAD,), jnp.float32),
    ],
)
def _sc_call(*refs):
  _sc_body(*refs)


def kernel(x, edge_index, W1, b1, W2, b2):
  x_pad = jnp.pad(x.astype(jnp.float32), ((0, 0), (0, N_PAD - NN)))
  src = edge_index[0].astype(jnp.int32)
  dst = edge_index[1].astype(jnp.int32)
  src = jnp.pad(src, (0, E_PAD - EE), constant_values=NN)
  dst = jnp.pad(dst, (0, E_PAD - EE), constant_values=NN)
  w1 = W1.reshape(-1).astype(jnp.float32)
  b1v = b1.reshape(-1).astype(jnp.float32)
  w2 = W2.reshape(-1).astype(jnp.float32)
  b2v = jnp.pad(b2.reshape(-1).astype(jnp.float32), (0, L - 1))
  out_flat = _sc_call(x_pad.reshape(-1), src, dst, w1, b1v, w2, b2v)
  return out_flat.reshape(BB, N_PAD)[:, :NN]


# speedup vs baseline: 152.3645x; 1.0033x over previous
import functools

import jax
import jax.numpy as jnp
from jax import lax
from jax.experimental import pallas as pl
from jax.experimental.pallas import tpu as pltpu
from jax.experimental.pallas import tpu_sc as plsc

NN = 50000
EE = 800000
BB = 4
HH = 64
NC, NS = 2, 16
L = 16
N_PAD = 50176
SEG = N_PAD // NS
CHUNK = 6400
E_PAD = 819200
WPT = E_PAD // NS
NCH = WPT // CHUNK


def _sc_body(x_hbm, src_hbm, dst_hbm, w1_hbm, b1_hbm, w2_hbm, b2_hbm,
             out_hbm,
             srcb0, srcb1, srcb2, dstb0, dstb1, dstb2, dstb3,
             valsa0, valsa1, valsa2, valsb0, valsb1, valsb2, ones,
             tmpa, tmpb, tmpc, tmpd,
             w1s, b1s, w2s, b2s,
             sem_pre, sem_g0, sem_g1, sem_s0, sem_s1,
             hist_sh, dinv_sh, u0_sh, u1_sh, a0_sh, a1_sh):
  srcs = [srcb0, srcb1, srcb2]
  dsts = [dstb0, dstb1, dstb2, dstb3]
  valsa = [valsa0, valsa1, valsa2]
  valsb = [valsb0, valsb1, valsb2]
  c = lax.axis_index("c")
  s = lax.axis_index("s")
  seg = s * SEG

  zeros16 = jnp.zeros((L,), jnp.float32)
  ones16 = jnp.ones((L,), jnp.float32)

  pltpu.sync_copy(w1_hbm, w1s)
  pltpu.sync_copy(b1_hbm, b1s)
  pltpu.sync_copy(w2_hbm, w2s)
  pltpu.sync_copy(b2_hbm, b2s)
  w1vs = [w1s[pl.ds(k * L, L)] for k in range(HH // L)]
  b1vs = [b1s[pl.ds(k * L, L)] for k in range(HH // L)]
  w2vs = [w2s[pl.ds(k * L, L)] for k in range(HH // L)]
  pacc = jnp.zeros((L,), jnp.float32)
  qacc = jnp.zeros((L,), jnp.float32)
  babs = jnp.zeros((L,), jnp.float32)
  for k in range(HH // L):
    pw = w1vs[k] * w2vs[k]
    pacc = pacc + jnp.where(w1vs[k] > 0.0, pw, 0.0)
    qacc = qacc + jnp.where(w1vs[k] < 0.0, pw, 0.0)
    babs = jnp.maximum(babs, jnp.abs(b1vs[k]))
  p_sum = jnp.sum(pacc)
  q_sum = jnp.sum(qacc)
  b1_is_zero = jnp.max(babs) == 0.0
  b2v = b2s[pl.ds(0, L)][0]

  @pl.loop(0, CHUNK // L)
  def _(i):
    ones[pl.ds(i * L, L)] = ones16

  @pl.loop(0, SEG // L)
  def _(i):
    tmpa[pl.ds(i * L, L)] = zeros16

  pltpu.sync_copy(tmpa, hist_sh.at[pl.ds(seg, SEG)])
  pltpu.sync_copy(tmpa, a0_sh.at[pl.ds(seg, SEG)])
  pltpu.sync_copy(tmpa, a1_sh.at[pl.ds(seg, SEG)])
  plsc.subcore_barrier()

  hpend = []
  pres = [pltpu.async_copy(dst_hbm.at[pl.ds(s * WPT, CHUNK)], dsts[0],
                           sem_pre)]
  for ch in range(NCH):
    if ch >= 3:
      hpend[ch - 3].wait()
    if ch + 1 < NCH:
      pres.append(pltpu.async_copy(
          dst_hbm.at[pl.ds(s * WPT + (ch + 1) * CHUNK, CHUNK)],
          dsts[(ch + 1) % 4], sem_pre))
    pres[ch].wait()
    hpend.append(pltpu.async_copy(ones, hist_sh.at[dsts[ch % 4]], sem_s0,
                                  add=True))
  for ch in range(max(0, NCH - 3), NCH):
    hpend[ch].wait()
  plsc.subcore_barrier()

  pltpu.sync_copy(hist_sh.at[pl.ds(seg, SEG)], tmpd)

  @pl.loop(0, SEG // L)
  def _(i):
    deg = tmpd[pl.ds(i * L, L)] + 1.0
    ibits = plsc.bitcast(deg, jnp.int32)
    y = plsc.bitcast(jnp.int32(0x5F3759DF) - (ibits >> 1), jnp.float32)
    half = deg * 0.5
    y = y * (1.5 - half * y * y)
    y = y * (1.5 - half * y * y)
    y = y * (1.5 - half * y * y)
    tmpd[pl.ds(i * L, L)] = y

  pltpu.sync_copy(tmpd, dinv_sh.at[pl.ds(seg, SEG)])

  for smp, u_sh in ((0, u0_sh), (1, u1_sh)):
    bs = 2 * c + smp
    pltpu.sync_copy(x_hbm.at[pl.ds(bs * N_PAD + seg, SEG)], tmpa)

    @pl.loop(0, SEG // L)
    def _(i):
      tmpa[pl.ds(i * L, L)] *= tmpd[pl.ds(i * L, L)]

    pltpu.sync_copy(tmpa, u_sh.at[pl.ds(seg, SEG)])

  plsc.subcore_barrier()

  def edge_pass():
    sspend = []
    pres = [(pltpu.async_copy(src_hbm.at[pl.ds(s * WPT, CHUNK)], srcs[0],
                              sem_pre),
             pltpu.async_copy(dst_hbm.at[pl.ds(s * WPT, CHUNK)], dsts[0],
                              sem_pre))]
    for ch in range(NCH):
      p3 = ch % 3
      p2 = ch % 2
      if ch >= 2:
        s0d, s1d = sspend[ch - 2]
        s0d.wait()
        s1d.wait()
      if ch + 1 < NCH:
        woff = s * WPT + (ch + 1) * CHUNK
        q3 = (ch + 1) % 3
        pres.append((pltpu.async_copy(src_hbm.at[pl.ds(woff, CHUNK)],
                                      srcs[q3], sem_pre),
                     pltpu.async_copy(dst_hbm.at[pl.ds(woff, CHUNK)],
                                      dsts[q3], sem_pre)))
      pa, pb = pres[ch]
      pa.wait()
      pb.wait()
      g0 = pltpu.async_copy(u0_sh.at[srcs[p3]], valsa[p2], sem_g0)
      g1 = pltpu.async_copy(u1_sh.at[srcs[p3]], valsb[p2], sem_g1)
      g0.wait()
      s0 = pltpu.async_copy(valsa[p2], a0_sh.at[dsts[p3]], sem_s0, add=True)
      g1.wait()
      s1 = pltpu.async_copy(valsb[p2], a1_sh.at[dsts[p3]], sem_s1, add=True)
      sspend.append((s0, s1))
    for ch in (NCH - 2, NCH - 1):
      s0d, s1d = sspend[ch]
      s0d.wait()
      s1d.wait()

  edge_pass()
  plsc.subcore_barrier()

  def dense(u_sh, a_sh):
    pltpu.sync_copy(a_sh.at[pl.ds(seg, SEG)], tmpa)
    pltpu.sync_copy(u_sh.at[pl.ds(seg, SEG)], tmpb)

    def fast(_):
      @pl.loop(0, SEG // L)
      def _(i):
        dv = tmpd[pl.ds(i * L, L)]
        s1 = dv * (tmpa[pl.ds(i * L, L)] + tmpb[pl.ds(i * L, L)])
        t = s1 * jnp.where(s1 > 0.0, p_sum, q_sum)
        tmpa[pl.ds(i * L, L)] = dv * t

    def full(_):
      @pl.loop(0, SEG // L)
      def _(i):
        dv = tmpd[pl.ds(i * L, L)]
        s1 = dv * (tmpa[pl.ds(i * L, L)] + tmpb[pl.ds(i * L, L)])
        t = jnp.zeros((L,), jnp.float32)
        for k in range(HH // L):
          for j in range(L):
            t = t + jnp.maximum(s1 * w1vs[k][j] + b1vs[k][j], 0.0) * w2vs[k][j]
        tmpa[pl.ds(i * L, L)] = dv * t

    lax.cond(b1_is_zero, fast, full, 0)
    pltpu.sync_copy(tmpa, u_sh.at[pl.ds(seg, SEG)])
    @pl.loop(0, SEG // L)
    def _(i):
      tmpb[pl.ds(i * L, L)] = zeros16

    pltpu.sync_copy(tmpb, a_sh.at[pl.ds(seg, SEG)])

  dense(u0_sh, a0_sh)
  dense(u1_sh, a1_sh)
  plsc.subcore_barrier()

  edge_pass()
  plsc.subcore_barrier()

  for smp, (u_sh, a_sh) in ((0, (u0_sh, a0_sh)), (1, (u1_sh, a1_sh))):
    bs = 2 * c + smp
    pltpu.sync_copy(a_sh.at[pl.ds(seg, SEG)], tmpa)
    pltpu.sync_copy(u_sh.at[pl.ds(seg, SEG)], tmpb)
    pltpu.sync_copy(x_hbm.at[pl.ds(bs * N_PAD + seg, SEG)], tmpc)

    @pl.loop(0, SEG // L)
    def _(i):
      dv = tmpd[pl.ds(i * L, L)]
      g2 = tmpa[pl.ds(i * L, L)]
      v = tmpb[pl.ds(i * L, L)]
      xv = tmpc[pl.ds(i * L, L)]
      tmpa[pl.ds(i * L, L)] = xv + 0.5 * (dv * (g2 + v) + b2v)

    pltpu.sync_copy(tmpa, out_hbm.at[pl.ds(bs * N_PAD + seg, SEG)])


@functools.partial(
    pl.kernel,
    out_type=jax.ShapeDtypeStruct((BB * N_PAD,), jnp.float32),
    mesh=plsc.VectorSubcoreMesh(
        core_axis_name="c", subcore_axis_name="s",
        num_cores=NC, num_subcores=NS),
    compiler_params=pltpu.CompilerParams(needs_layout_passes=False),
    scratch_types=[
        pltpu.VMEM((CHUNK,), jnp.int32),
        pltpu.VMEM((CHUNK,), jnp.int32),
        pltpu.VMEM((CHUNK,), jnp.int32),
        pltpu.VMEM((CHUNK,), jnp.int32),
        pltpu.VMEM((CHUNK,), jnp.int32),
        pltpu.VMEM((CHUNK,), jnp.int32),
        pltpu.VMEM((CHUNK,), jnp.int32),
        pltpu.VMEM((CHUNK,), jnp.float32),
        pltpu.VMEM((CHUNK,), jnp.float32),
        pltpu.VMEM((CHUNK,), jnp.float32),
        pltpu.VMEM((CHUNK,), jnp.float32),
        pltpu.VMEM((CHUNK,), jnp.float32),
        pltpu.VMEM((CHUNK,), jnp.float32),
        pltpu.VMEM((CHUNK,), jnp.float32),
        pltpu.VMEM((SEG,), jnp.float32),
        pltpu.VMEM((SEG,), jnp.float32),
        pltpu.VMEM((SEG,), jnp.float32),
        pltpu.VMEM((SEG,), jnp.float32),
        pltpu.VMEM((HH,), jnp.float32),
        pltpu.VMEM((HH,), jnp.float32),
        pltpu.VMEM((HH,), jnp.float32),
        pltpu.VMEM((L,), jnp.float32),
        pltpu.SemaphoreType.DMA,
        pltpu.SemaphoreType.DMA,
        pltpu.SemaphoreType.DMA,
        pltpu.SemaphoreType.DMA,
        pltpu.SemaphoreType.DMA,
        pltpu.VMEM_SHARED((N_PAD,), jnp.float32),
        pltpu.VMEM_SHARED((N_PAD,), jnp.float32),
        pltpu.VMEM_SHARED((N_PAD,), jnp.float32),
        pltpu.VMEM_SHARED((N_PAD,), jnp.float32),
        pltpu.VMEM_SHARED((N_PAD,), jnp.float32),
        pltpu.VMEM_SHARED((N_PAD,), jnp.float32),
    ],
)
def _sc_call(*refs):
  _sc_body(*refs)


def kernel(x, edge_index, W1, b1, W2, b2):
  x_pad = jnp.pad(x.astype(jnp.float32), ((0, 0), (0, N_PAD - NN)))
  src = edge_index[0].astype(jnp.int32)
  dst = edge_index[1].astype(jnp.int32)
  src = jnp.pad(src, (0, E_PAD - EE), constant_values=NN)
  dst = jnp.pad(dst, (0, E_PAD - EE), constant_values=NN)
  w1 = W1.reshape(-1).astype(jnp.float32)
  b1v = b1.reshape(-1).astype(jnp.float32)
  w2 = W2.reshape(-1).astype(jnp.float32)
  b2v = jnp.pad(b2.reshape(-1).astype(jnp.float32), (0, L - 1))
  out_flat = _sc_call(x_pad.reshape(-1), src, dst, w1, b1v, w2, b2v)
  return out_flat.reshape(BB, N_PAD)[:, :NN]


# speedup vs baseline: 154.5337x; 1.0142x over previous
import functools

import jax
import jax.numpy as jnp
from jax import lax
from jax.experimental import pallas as pl
from jax.experimental.pallas import tpu as pltpu
from jax.experimental.pallas import tpu_sc as plsc

NN = 50000
EE = 800000
BB = 4
HH = 64
NC, NS = 2, 16
L = 16
N_PAD = 50176
SEG = N_PAD // NS
CHUNK = 6400
E_PAD = 819200
WPT = E_PAD // NS
NCH = WPT // CHUNK


def _sc_body(x_hbm, src_hbm, dst_hbm, w1_hbm, b1_hbm, w2_hbm, b2_hbm,
             out_hbm,
             srcb0, srcb1, srcb2, dstb0, dstb1, dstb2, dstb3,
             valsa0, valsa1, valsa2, valsb0, valsb1, valsb2, ones,
             tmpa, tmpb, tmpc, tmpd,
             w1s, b1s, w2s, b2s,
             sem_pre, sem_g0, sem_g1, sem_s0, sem_s1,
             hist_sh, dinv_sh, u0_sh, u1_sh, a0_sh, a1_sh):
  srcs = [srcb0, srcb1, srcb2]
  dsts = [dstb0, dstb1, dstb2, dstb3]
  valsa = [valsa0, valsa1, valsa2]
  valsb = [valsb0, valsb1, valsb2]
  c = lax.axis_index("c")
  s = lax.axis_index("s")
  seg = s * SEG

  zeros16 = jnp.zeros((L,), jnp.float32)
  ones16 = jnp.ones((L,), jnp.float32)

  pltpu.sync_copy(w1_hbm, w1s)
  pltpu.sync_copy(b1_hbm, b1s)
  pltpu.sync_copy(w2_hbm, w2s)
  pltpu.sync_copy(b2_hbm, b2s)
  w1vs = [w1s[pl.ds(k * L, L)] for k in range(HH // L)]
  b1vs = [b1s[pl.ds(k * L, L)] for k in range(HH // L)]
  w2vs = [w2s[pl.ds(k * L, L)] for k in range(HH // L)]
  pacc = jnp.zeros((L,), jnp.float32)
  qacc = jnp.zeros((L,), jnp.float32)
  babs = jnp.zeros((L,), jnp.float32)
  for k in range(HH // L):
    pw = w1vs[k] * w2vs[k]
    pacc = pacc + jnp.where(w1vs[k] > 0.0, pw, 0.0)
    qacc = qacc + jnp.where(w1vs[k] < 0.0, pw, 0.0)
    babs = jnp.maximum(babs, jnp.abs(b1vs[k]))
  p_sum = jnp.sum(pacc)
  q_sum = jnp.sum(qacc)
  b1_is_zero = jnp.max(babs) == 0.0
  b2v = b2s[pl.ds(0, L)][0]

  @pl.loop(0, CHUNK // L, unroll=8)
  def _(i):
    ones[pl.ds(i * L, L)] = ones16

  @pl.loop(0, SEG // L, unroll=8)
  def _(i):
    tmpa[pl.ds(i * L, L)] = zeros16

  pltpu.sync_copy(tmpa, hist_sh.at[pl.ds(seg, SEG)])
  pltpu.sync_copy(tmpa, a0_sh.at[pl.ds(seg, SEG)])
  pltpu.sync_copy(tmpa, a1_sh.at[pl.ds(seg, SEG)])
  plsc.subcore_barrier()

  hpend = []
  pres = [pltpu.async_copy(dst_hbm.at[pl.ds(s * WPT, CHUNK)], dsts[0],
                           sem_pre)]
  for ch in range(NCH):
    if ch >= 3:
      hpend[ch - 3].wait()
    if ch + 1 < NCH:
      pres.append(pltpu.async_copy(
          dst_hbm.at[pl.ds(s * WPT + (ch + 1) * CHUNK, CHUNK)],
          dsts[(ch + 1) % 4], sem_pre))
    pres[ch].wait()
    hpend.append(pltpu.async_copy(ones, hist_sh.at[dsts[ch % 4]], sem_s0,
                                  add=True))
  for ch in range(max(0, NCH - 3), NCH):
    hpend[ch].wait()
  plsc.subcore_barrier()

  pltpu.sync_copy(hist_sh.at[pl.ds(seg, SEG)], tmpd)

  @pl.loop(0, SEG // L, unroll=4)
  def _(i):
    deg = tmpd[pl.ds(i * L, L)] + 1.0
    ibits = plsc.bitcast(deg, jnp.int32)
    y = plsc.bitcast(jnp.int32(0x5F3759DF) - (ibits >> 1), jnp.float32)
    half = deg * 0.5
    y = y * (1.5 - half * y * y)
    y = y * (1.5 - half * y * y)
    y = y * (1.5 - half * y * y)
    tmpd[pl.ds(i * L, L)] = y

  pltpu.sync_copy(tmpd, dinv_sh.at[pl.ds(seg, SEG)])

  for smp, u_sh in ((0, u0_sh), (1, u1_sh)):
    bs = 2 * c + smp
    pltpu.sync_copy(x_hbm.at[pl.ds(bs * N_PAD + seg, SEG)], tmpa)

    @pl.loop(0, SEG // L, unroll=8)
    def _(i):
      tmpa[pl.ds(i * L, L)] *= tmpd[pl.ds(i * L, L)]

    pltpu.sync_copy(tmpa, u_sh.at[pl.ds(seg, SEG)])

  plsc.subcore_barrier()

  def edge_pass():
    sspend = []
    pres = [(pltpu.async_copy(src_hbm.at[pl.ds(s * WPT, CHUNK)], srcs[0],
                              sem_pre),
             pltpu.async_copy(dst_hbm.at[pl.ds(s * WPT, CHUNK)], dsts[0],
                              sem_pre))]
    for ch in range(NCH):
      p3 = ch % 3
      p2 = ch % 2
      if ch >= 2:
        s0d, s1d = sspend[ch - 2]
        s0d.wait()
        s1d.wait()
      if ch + 1 < NCH:
        woff = s * WPT + (ch + 1) * CHUNK
        q3 = (ch + 1) % 3
        pres.append((pltpu.async_copy(src_hbm.at[pl.ds(woff, CHUNK)],
                                      srcs[q3], sem_pre),
                     pltpu.async_copy(dst_hbm.at[pl.ds(woff, CHUNK)],
                                      dsts[q3], sem_pre)))
      pa, pb = pres[ch]
      pa.wait()
      pb.wait()
      g0 = pltpu.async_copy(u0_sh.at[srcs[p3]], valsa[p2], sem_g0)
      g1 = pltpu.async_copy(u1_sh.at[srcs[p3]], valsb[p2], sem_g1)
      g0.wait()
      s0 = pltpu.async_copy(valsa[p2], a0_sh.at[dsts[p3]], sem_s0, add=True)
      g1.wait()
      s1 = pltpu.async_copy(valsb[p2], a1_sh.at[dsts[p3]], sem_s1, add=True)
      sspend.append((s0, s1))
    for ch in (NCH - 2, NCH - 1):
      s0d, s1d = sspend[ch]
      s0d.wait()
      s1d.wait()

  edge_pass()
  plsc.subcore_barrier()

  def dense(u_sh, a_sh):
    pltpu.sync_copy(a_sh.at[pl.ds(seg, SEG)], tmpa)
    pltpu.sync_copy(u_sh.at[pl.ds(seg, SEG)], tmpb)

    def fast(_):
      @pl.loop(0, SEG // L, unroll=4)
      def _(i):
        dv = tmpd[pl.ds(i * L, L)]
        s1 = dv * (tmpa[pl.ds(i * L, L)] + tmpb[pl.ds(i * L, L)])
        t = s1 * jnp.where(s1 > 0.0, p_sum, q_sum)
        tmpa[pl.ds(i * L, L)] = dv * t

    def full(_):
      @pl.loop(0, SEG // L)
      def _(i):
        dv = tmpd[pl.ds(i * L, L)]
        s1 = dv * (tmpa[pl.ds(i * L, L)] + tmpb[pl.ds(i * L, L)])
        t = jnp.zeros((L,), jnp.float32)
        for k in range(HH // L):
          for j in range(L):
            t = t + jnp.maximum(s1 * w1vs[k][j] + b1vs[k][j], 0.0) * w2vs[k][j]
        tmpa[pl.ds(i * L, L)] = dv * t

    lax.cond(b1_is_zero, fast, full, 0)
    pltpu.sync_copy(tmpa, u_sh.at[pl.ds(seg, SEG)])
    @pl.loop(0, SEG // L, unroll=8)
    def _(i):
      tmpb[pl.ds(i * L, L)] = zeros16

    pltpu.sync_copy(tmpb, a_sh.at[pl.ds(seg, SEG)])

  dense(u0_sh, a0_sh)
  dense(u1_sh, a1_sh)
  plsc.subcore_barrier()

  edge_pass()
  plsc.subcore_barrier()

  for smp, (u_sh, a_sh) in ((0, (u0_sh, a0_sh)), (1, (u1_sh, a1_sh))):
    bs = 2 * c + smp
    pltpu.sync_copy(a_sh.at[pl.ds(seg, SEG)], tmpa)
    pltpu.sync_copy(u_sh.at[pl.ds(seg, SEG)], tmpb)
    pltpu.sync_copy(x_hbm.at[pl.ds(bs * N_PAD + seg, SEG)], tmpc)

    @pl.loop(0, SEG // L, unroll=4)
    def _(i):
      dv = tmpd[pl.ds(i * L, L)]
      g2 = tmpa[pl.ds(i * L, L)]
      v = tmpb[pl.ds(i * L, L)]
      xv = tmpc[pl.ds(i * L, L)]
      tmpa[pl.ds(i * L, L)] = xv + 0.5 * (dv * (g2 + v) + b2v)

    pltpu.sync_copy(tmpa, out_hbm.at[pl.ds(bs * N_PAD + seg, SEG)])


@functools.partial(
    pl.kernel,
    out_type=jax.ShapeDtypeStruct((BB * N_PAD,), jnp.float32),
    mesh=plsc.VectorSubcoreMesh(
        core_axis_name="c", subcore_axis_name="s",
        num_cores=NC, num_subcores=NS),
    compiler_params=pltpu.CompilerParams(needs_layout_passes=False),
    scratch_types=[
        pltpu.VMEM((CHUNK,), jnp.int32),
        pltpu.VMEM((CHUNK,), jnp.int32),
        pltpu.VMEM((CHUNK,), jnp.int32),
        pltpu.VMEM((CHUNK,), jnp.int32),
        pltpu.VMEM((CHUNK,), jnp.int32),
        pltpu.VMEM((CHUNK,), jnp.int32),
        pltpu.VMEM((CHUNK,), jnp.int32),
        pltpu.VMEM((CHUNK,), jnp.float32),
        pltpu.VMEM((CHUNK,), jnp.float32),
        pltpu.VMEM((CHUNK,), jnp.float32),
        pltpu.VMEM((CHUNK,), jnp.float32),
        pltpu.VMEM((CHUNK,), jnp.float32),
        pltpu.VMEM((CHUNK,), jnp.float32),
        pltpu.VMEM((CHUNK,), jnp.float32),
        pltpu.VMEM((SEG,), jnp.float32),
        pltpu.VMEM((SEG,), jnp.float32),
        pltpu.VMEM((SEG,), jnp.float32),
        pltpu.VMEM((SEG,), jnp.float32),
        pltpu.VMEM((HH,), jnp.float32),
        pltpu.VMEM((HH,), jnp.float32),
        pltpu.VMEM((HH,), jnp.float32),
        pltpu.VMEM((L,), jnp.float32),
        pltpu.SemaphoreType.DMA,
        pltpu.SemaphoreType.DMA,
        pltpu.SemaphoreType.DMA,
        pltpu.SemaphoreType.DMA,
        pltpu.SemaphoreType.DMA,
        pltpu.VMEM_SHARED((N_PAD,), jnp.float32),
        pltpu.VMEM_SHARED((N_PAD,), jnp.float32),
        pltpu.VMEM_SHARED((N_PAD,), jnp.float32),
        pltpu.VMEM_SHARED((N_PAD,), jnp.float32),
        pltpu.VMEM_SHARED((N_PAD,), jnp.float32),
        pltpu.VMEM_SHARED((N_PAD,), jnp.float32),
    ],
)
def _sc_call(*refs):
  _sc_body(*refs)


def kernel(x, edge_index, W1, b1, W2, b2):
  x_pad = jnp.pad(x.astype(jnp.float32), ((0, 0), (0, N_PAD - NN)))
  src = edge_index[0].astype(jnp.int32)
  dst = edge_index[1].astype(jnp.int32)
  src = jnp.pad(src, (0, E_PAD - EE), constant_values=NN)
  dst = jnp.pad(dst, (0, E_PAD - EE), constant_values=NN)
  w1 = W1.reshape(-1).astype(jnp.float32)
  b1v = b1.reshape(-1).astype(jnp.float32)
  w2 = W2.reshape(-1).astype(jnp.float32)
  b2v = jnp.pad(b2.reshape(-1).astype(jnp.float32), (0, L - 1))
  out_flat = _sc_call(x_pad.reshape(-1), src, dst, w1, b1v, w2, b2v)
  return out_flat.reshape(BB, N_PAD)[:, :NN]
